# baseline (speedup 1.0000x reference)
import functools

import jax
import jax.numpy as jnp
from jax import lax
from jax.experimental import pallas as pl
from jax.experimental.pallas import tpu as pltpu
from jax.experimental.pallas import tpu_sc as plsc

N = 10000
E = 320000
F_IN = 128
H = 32

NC = 2
NS = 16
NW = NC * NS
CHUNK = 128
KCH = 80
GRP = 8
NG = KCH // GRP
EPW = KCH * CHUNK
EPAD = NW * EPW
NPAD = 10112
ROWS_PER_TILE = NPAD // NS


_MESH = plsc.VectorSubcoreMesh(core_axis_name="c", subcore_axis_name="s")
_SC_PARAMS = pltpu.CompilerParams(use_tc_tiling_on_sc=False)


@functools.partial(
    pl.kernel,
    out_type=jax.ShapeDtypeStruct((NC, NPAD, H), jnp.float32),
    mesh=_MESH,
    compiler_params=_SC_PARAMS,
    scratch_types=[
        pltpu.VMEM((KCH, CHUNK), jnp.int32),
        pltpu.VMEM((CHUNK, H), jnp.float32),
        pltpu.VMEM_SHARED((NPAD, H), jnp.float32),
        pltpu.SemaphoreType.DMA,
    ],
)
def _sc_degree(dst_hbm, ones_hbm, zeros_hbm, out_hbm, dst_v, ones_v, deg_s, ssem):
    cid = lax.axis_index("c")
    sid = lax.axis_index("s")
    wid = cid * NS + sid

    pltpu.sync_copy(dst_hbm.at[wid], dst_v)
    pltpu.sync_copy(ones_hbm, ones_v)
    pltpu.sync_copy(
        zeros_hbm.at[pl.ds(sid * ROWS_PER_TILE, ROWS_PER_TILE)],
        deg_s.at[pl.ds(sid * ROWS_PER_TILE, ROWS_PER_TILE)],
    )
    plsc.subcore_barrier()

    def fire(j, carry):
        pltpu.async_copy(ones_v, deg_s.at[dst_v.at[j]], ssem, add=True)
        return carry

    def drain(j, carry):
        pltpu.make_async_copy(ones_v, deg_s.at[dst_v.at[j]], ssem).wait()
        return carry

    lax.fori_loop(0, KCH, fire, 0)
    lax.fori_loop(0, KCH, drain, 0)
    plsc.subcore_barrier()

    pltpu.sync_copy(
        deg_s.at[pl.ds(sid * ROWS_PER_TILE, ROWS_PER_TILE)],
        out_hbm.at[cid, pl.ds(sid * ROWS_PER_TILE, ROWS_PER_TILE)],
    )


@functools.partial(
    pl.kernel,
    out_type=jax.ShapeDtypeStruct((NC, NPAD, H), jnp.float32),
    mesh=_MESH,
    compiler_params=_SC_PARAMS,
    scratch_types=[
        pltpu.VMEM((KCH, CHUNK), jnp.int32),
        pltpu.VMEM((KCH, CHUNK), jnp.int32),
        pltpu.VMEM((2, GRP, CHUNK, H), jnp.float32),
        pltpu.VMEM_SHARED((NPAD, H), jnp.float32),
        pltpu.VMEM_SHARED((NPAD, H), jnp.float32),
        pltpu.SemaphoreType.DMA,
        pltpu.SemaphoreType.DMA,
    ],
)
def _sc_aggregate(table_hbm, src_hbm, dst_hbm, zeros_hbm, out_hbm,
                  src_v, dst_v, rows_v, acc_s, table_s, gsem, ssem):
    cid = lax.axis_index("c")
    sid = lax.axis_index("s")
    wid = cid * NS + sid

    pltpu.sync_copy(src_hbm.at[wid], src_v)
    pltpu.sync_copy(dst_hbm.at[wid], dst_v)
    sl = pl.ds(sid * ROWS_PER_TILE, ROWS_PER_TILE)

    @pl.when(cid == 0)
    def _():
        pltpu.sync_copy(table_hbm.at[sl], acc_s.at[sl])

    @pl.when(cid != 0)
    def _():
        pltpu.sync_copy(zeros_hbm.at[sl], acc_s.at[sl])

    pltpu.sync_copy(table_hbm.at[sl], table_s.at[sl])
    plsc.subcore_barrier()

    def fire_gathers(g, p):
        for b in range(GRP):
            pltpu.async_copy(table_s.at[src_v.at[g * GRP + b]],
                             rows_v.at[p, b], gsem)

    def drain_gathers(p):
        for b in range(GRP):
            pltpu.make_async_copy(table_s.at[src_v.at[0]],
                                  rows_v.at[p, b], gsem).wait()

    def fire_scatters(g, p):
        for b in range(GRP):
            pltpu.async_copy(rows_v.at[p, b],
                             acc_s.at[dst_v.at[g * GRP + b]], ssem, add=True)

    def drain_scatters(g, p):
        for b in range(GRP):
            pltpu.make_async_copy(rows_v.at[p, b],
                                  acc_s.at[dst_v.at[g * GRP + b]], ssem).wait()

    fire_gathers(0, 0)
    fire_gathers(1, 1)

    def body(k, carry):
        g0 = 2 * k
        g1 = g0 + 1
        drain_gathers(0)
        fire_scatters(g0, 0)
        drain_gathers(1)
        fire_scatters(g1, 1)
        drain_scatters(g0, 0)

        @pl.when(g0 + 2 < NG)
        def _():
            fire_gathers(g0 + 2, 0)

        drain_scatters(g1, 1)

        @pl.when(g1 + 2 < NG)
        def _():
            fire_gathers(g1 + 2, 1)

        return carry

    lax.fori_loop(0, NG // 2, body, 0)
    plsc.subcore_barrier()

    pltpu.sync_copy(
        acc_s.at[pl.ds(sid * ROWS_PER_TILE, ROWS_PER_TILE)],
        out_hbm.at[cid, pl.ds(sid * ROWS_PER_TILE, ROWS_PER_TILE)],
    )


NP4 = NPAD // 4


_PREP_GRID = 4
_PREP_ROWS = NPAD // _PREP_GRID
_PREP_P4 = NP4 // _PREP_GRID


def _tc_prep_body(x_ref, w1_ref, degp_ref, h1p_ref, dinv_ref):
    dinv = lax.rsqrt(1.0 + degp_ref[0] + degp_ref[1])
    h = jnp.dot(x_ref[...], w1_ref[...], preferred_element_type=jnp.float32)
    h3 = jnp.reshape(h, (_PREP_P4, 4, H))
    for k in range(4):
        h1p_ref[:, k * H:(k + 1) * H] = h3[:, k, :] * dinv[:, k * H:(k + 1) * H]
    dinv_ref[...] = dinv


def _tc_prep(x, w1, degp):
    return pl.pallas_call(
        _tc_prep_body,
        grid=(_PREP_GRID,),
        in_specs=[
            pl.BlockSpec((_PREP_ROWS, F_IN), lambda i: (i, 0)),
            pl.BlockSpec((F_IN, H), lambda i: (0, 0)),
            pl.BlockSpec((NC, _PREP_P4, 128), lambda i: (0, i, 0)),
        ],
        out_specs=(
            pl.BlockSpec((_PREP_P4, 128), lambda i: (i, 0)),
            pl.BlockSpec((_PREP_P4, 128), lambda i: (i, 0)),
        ),
        out_shape=(
            jax.ShapeDtypeStruct((NP4, 128), jnp.float32),
            jax.ShapeDtypeStruct((NP4, 128), jnp.float32),
        ),
    )(x, w1, degp)


def _tc_mid_body(accp_ref, dinv_ref, w2p_ref, b1p_ref, h2p_ref):
    dinv = dinv_ref[...]
    acc = accp_ref[0] + accp_ref[1]
    h1 = jnp.maximum(acc * dinv + b1p_ref[...], 0.0)
    g = jnp.dot(h1, w2p_ref[...], preferred_element_type=jnp.float32)
    h2p_ref[...] = g * dinv


def _tc_mid(accp, dinv, w2p, b1p):
    return pl.pallas_call(
        _tc_mid_body,
        out_shape=jax.ShapeDtypeStruct((NP4, 128), jnp.float32),
    )(accp, dinv, w2p, b1p)


def _tc_final_body(accp_ref, dinv_ref, w3p_ref, b2p_ref, b3p_ref, out_ref):
    dinv = dinv_ref[...]
    acc = accp_ref[0] + accp_ref[1]
    h2 = jnp.maximum(acc * dinv + b2p_ref[...], 0.0)
    out_ref[...] = jnp.dot(
        h2, w3p_ref[...], preferred_element_type=jnp.float32) + b3p_ref[...]


def _tc_final(accp, dinv, w3p, b2p, b3p):
    return pl.pallas_call(
        _tc_final_body,
        out_shape=jax.ShapeDtypeStruct((NP4, 4), jnp.float32),
    )(accp, dinv, w3p, b2p, b3p)


@jax.jit
def kernel(x, edge_index, W1, b1, W2, b2, W3, b3):
    pad_e = EPAD - E
    src = jnp.concatenate(
        [edge_index[0], jnp.zeros((pad_e,), jnp.int32)]).reshape(NW, KCH, CHUNK)
    dst = jnp.concatenate(
        [edge_index[1], jnp.full((pad_e,), N, jnp.int32)]).reshape(NW, KCH, CHUNK)

    ones_rows = jnp.ones((CHUNK, H), jnp.float32)
    zeros_h = jnp.zeros((NPAD, H), jnp.float32)
    eye4 = jnp.eye(4, dtype=jnp.float32)
    w2p = (eye4[:, None, :, None] * W2[None, :, None, :]).reshape(128, 128)
    w3p = (eye4[:, None, :, None] * W3[None, :, None, :]).reshape(128, 4)
    b1p = jnp.tile(b1, 4).reshape(1, 128)
    b2p = jnp.tile(b2, 4).reshape(1, 128)
    b3p = jnp.tile(b3, 4).reshape(1, 4)

    degp = _sc_degree(dst, ones_rows, zeros_h)
    x_pad = jnp.pad(x, ((0, NPAD - N), (0, 0)))
    h1p, dinv = _tc_prep(x_pad, W1, degp.reshape(NC, NP4, 128))
    acc1 = _sc_aggregate(h1p.reshape(NPAD, H), src, dst, zeros_h)
    h2p = _tc_mid(acc1.reshape(NC, NP4, 128), dinv, w2p, b1p)
    acc2 = _sc_aggregate(h2p.reshape(NPAD, H), src, dst, zeros_h)
    outp = _tc_final(acc2.reshape(NC, NP4, 128), dinv, w3p, b2p, b3p)
    return outp.reshape(NPAD, 1)[:N]


# speedup vs baseline: 1.0537x; 1.0537x over previous
import functools

import jax
import jax.numpy as jnp
from jax import lax
from jax.experimental import pallas as pl
from jax.experimental.pallas import tpu as pltpu
from jax.experimental.pallas import tpu_sc as plsc

N = 10000
E = 320000
F_IN = 128
H = 32

NC = 2
NS = 16
NW = NC * NS
CHUNK = 128
KCH = 80
GRP = 8
NG = KCH // GRP
EPW = KCH * CHUNK
EPAD = NW * EPW
NPAD = 10112
ROWS_PER_TILE = NPAD // NS


_MESH = plsc.VectorSubcoreMesh(core_axis_name="c", subcore_axis_name="s")
_SC_PARAMS = pltpu.CompilerParams(use_tc_tiling_on_sc=False)


@functools.partial(
    pl.kernel,
    out_type=jax.ShapeDtypeStruct((NC, NPAD, H), jnp.float32),
    mesh=_MESH,
    compiler_params=_SC_PARAMS,
    scratch_types=[
        pltpu.VMEM((KCH, CHUNK), jnp.int32),
        pltpu.VMEM((CHUNK, H), jnp.float32),
        pltpu.VMEM_SHARED((NPAD, H), jnp.float32),
        pltpu.SemaphoreType.DMA,
    ],
)
def _sc_degree(dst_hbm, ones_hbm, zeros_hbm, out_hbm, dst_v, ones_v, deg_s, ssem):
    cid = lax.axis_index("c")
    sid = lax.axis_index("s")
    wid = cid * NS + sid

    pltpu.sync_copy(dst_hbm.at[wid], dst_v)
    pltpu.sync_copy(ones_hbm, ones_v)
    pltpu.sync_copy(
        zeros_hbm.at[pl.ds(sid * ROWS_PER_TILE, ROWS_PER_TILE)],
        deg_s.at[pl.ds(sid * ROWS_PER_TILE, ROWS_PER_TILE)],
    )
    plsc.subcore_barrier()

    def fire(j, carry):
        pltpu.async_copy(ones_v, deg_s.at[dst_v.at[j]], ssem, add=True)
        return carry

    def drain(j, carry):
        pltpu.make_async_copy(ones_v, deg_s.at[dst_v.at[j]], ssem).wait()
        return carry

    lax.fori_loop(0, KCH, fire, 0)
    lax.fori_loop(0, KCH, drain, 0)
    plsc.subcore_barrier()

    pltpu.sync_copy(
        deg_s.at[pl.ds(sid * ROWS_PER_TILE, ROWS_PER_TILE)],
        out_hbm.at[cid, pl.ds(sid * ROWS_PER_TILE, ROWS_PER_TILE)],
    )


@functools.partial(
    pl.kernel,
    out_type=jax.ShapeDtypeStruct((NC, NPAD, H), jnp.float32),
    mesh=_MESH,
    compiler_params=_SC_PARAMS,
    scratch_types=[
        pltpu.VMEM((KCH, CHUNK), jnp.int32),
        pltpu.VMEM((KCH, CHUNK), jnp.int32),
        pltpu.VMEM((2, GRP, CHUNK, H), jnp.float32),
        pltpu.VMEM_SHARED((NPAD, H), jnp.float32),
        pltpu.VMEM_SHARED((NPAD, H), jnp.float32),
        pltpu.SemaphoreType.DMA,
        pltpu.SemaphoreType.DMA,
    ],
)
def _sc_aggregate(table_hbm, src_hbm, dst_hbm, zeros_hbm, out_hbm,
                  src_v, dst_v, rows_v, acc_s, table_s, gsem, ssem):
    cid = lax.axis_index("c")
    sid = lax.axis_index("s")
    wid = cid * NS + sid

    pltpu.sync_copy(src_hbm.at[wid], src_v)
    pltpu.sync_copy(dst_hbm.at[wid], dst_v)
    sl = pl.ds(sid * ROWS_PER_TILE, ROWS_PER_TILE)

    @pl.when(cid == 0)
    def _():
        pltpu.sync_copy(table_hbm.at[sl], acc_s.at[sl])

    @pl.when(cid != 0)
    def _():
        pltpu.sync_copy(zeros_hbm.at[sl], acc_s.at[sl])

    pltpu.sync_copy(table_hbm.at[sl], table_s.at[sl])
    plsc.subcore_barrier()

    def fire_gathers(g, p):
        for b in range(GRP):
            pltpu.async_copy(table_s.at[src_v.at[g * GRP + b]],
                             rows_v.at[p, b], gsem)

    def drain_gathers(p):
        for b in range(GRP):
            pltpu.make_async_copy(table_s.at[src_v.at[0]],
                                  rows_v.at[p, b], gsem).wait()

    def fire_scatters(g, p):
        for b in range(GRP):
            pltpu.async_copy(rows_v.at[p, b],
                             acc_s.at[dst_v.at[g * GRP + b]], ssem, add=True)

    def drain_scatters(g, p):
        for b in range(GRP):
            pltpu.make_async_copy(rows_v.at[p, b],
                                  acc_s.at[dst_v.at[g * GRP + b]], ssem).wait()

    fire_gathers(0, 0)
    fire_gathers(1, 1)

    def body(k, carry):
        g0 = 2 * k
        g1 = g0 + 1
        drain_gathers(0)
        fire_scatters(g0, 0)
        drain_gathers(1)
        fire_scatters(g1, 1)
        drain_scatters(g0, 0)

        @pl.when(g0 + 2 < NG)
        def _():
            fire_gathers(g0 + 2, 0)

        drain_scatters(g1, 1)

        @pl.when(g1 + 2 < NG)
        def _():
            fire_gathers(g1 + 2, 1)

        return carry

    lax.fori_loop(0, NG // 2, body, 0)
    plsc.subcore_barrier()

    pltpu.sync_copy(
        acc_s.at[pl.ds(sid * ROWS_PER_TILE, ROWS_PER_TILE)],
        out_hbm.at[cid, pl.ds(sid * ROWS_PER_TILE, ROWS_PER_TILE)],
    )


NP4 = NPAD // 4


_PREP_GRID = 4
_PREP_ROWS = NPAD // _PREP_GRID
_PREP_P4 = NP4 // _PREP_GRID


def _tc_mm1_body(x_ref, w1_ref, hp_ref):
    h = jnp.dot(x_ref[...], w1_ref[...], preferred_element_type=jnp.float32)
    h3 = jnp.reshape(h, (_PREP_P4, 4, H))
    for k in range(4):
        hp_ref[:, k * H:(k + 1) * H] = h3[:, k, :]


def _tc_mm1(x, w1):
    return pl.pallas_call(
        _tc_mm1_body,
        grid=(_PREP_GRID,),
        in_specs=[
            pl.BlockSpec((_PREP_ROWS, F_IN), lambda i: (i, 0)),
            pl.BlockSpec((F_IN, H), lambda i: (0, 0)),
        ],
        out_specs=pl.BlockSpec((_PREP_P4, 128), lambda i: (i, 0)),
        out_shape=jax.ShapeDtypeStruct((NP4, 128), jnp.float32),
    )(x, w1)


def _tc_scale_body(hp_ref, degp_ref, h1p_ref, dinv_ref):
    dinv = lax.rsqrt(1.0 + degp_ref[0] + degp_ref[1])
    h1p_ref[...] = hp_ref[...] * dinv
    dinv_ref[...] = dinv


def _tc_scale(hp, degp):
    return pl.pallas_call(
        _tc_scale_body,
        out_shape=(
            jax.ShapeDtypeStruct((NP4, 128), jnp.float32),
            jax.ShapeDtypeStruct((NP4, 128), jnp.float32),
        ),
    )(hp, degp)


def _tc_mid_body(accp_ref, dinv_ref, w2p_ref, b1p_ref, h2p_ref):
    dinv = dinv_ref[...]
    acc = accp_ref[0] + accp_ref[1]
    h1 = jnp.maximum(acc * dinv + b1p_ref[...], 0.0)
    g = jnp.dot(h1, w2p_ref[...], preferred_element_type=jnp.float32)
    h2p_ref[...] = g * dinv


def _tc_mid(accp, dinv, w2p, b1p):
    return pl.pallas_call(
        _tc_mid_body,
        out_shape=jax.ShapeDtypeStruct((NP4, 128), jnp.float32),
    )(accp, dinv, w2p, b1p)


def _tc_final_body(accp_ref, dinv_ref, w3p_ref, b2p_ref, b3p_ref, out_ref):
    dinv = dinv_ref[...]
    acc = accp_ref[0] + accp_ref[1]
    h2 = jnp.maximum(acc * dinv + b2p_ref[...], 0.0)
    out_ref[...] = jnp.dot(
        h2, w3p_ref[...], preferred_element_type=jnp.float32) + b3p_ref[...]


def _tc_final(accp, dinv, w3p, b2p, b3p):
    return pl.pallas_call(
        _tc_final_body,
        out_shape=jax.ShapeDtypeStruct((NP4, 4), jnp.float32),
    )(accp, dinv, w3p, b2p, b3p)


@jax.jit
def kernel(x, edge_index, W1, b1, W2, b2, W3, b3):
    pad_e = EPAD - E
    src = jnp.concatenate(
        [edge_index[0], jnp.zeros((pad_e,), jnp.int32)]).reshape(NW, KCH, CHUNK)
    dst = jnp.concatenate(
        [edge_index[1], jnp.full((pad_e,), N, jnp.int32)]).reshape(NW, KCH, CHUNK)

    ones_rows = jnp.ones((CHUNK, H), jnp.float32)
    zeros_h = jnp.zeros((NPAD, H), jnp.float32)
    eye4 = jnp.eye(4, dtype=jnp.float32)
    w2p = (eye4[:, None, :, None] * W2[None, :, None, :]).reshape(128, 128)
    w3p = (eye4[:, None, :, None] * W3[None, :, None, :]).reshape(128, 4)
    b1p = jnp.tile(b1, 4).reshape(1, 128)
    b2p = jnp.tile(b2, 4).reshape(1, 128)
    b3p = jnp.tile(b3, 4).reshape(1, 4)

    x_pad = jnp.pad(x, ((0, NPAD - N), (0, 0)))
    degp = _sc_degree(dst, ones_rows, zeros_h)
    hp = _tc_mm1(x_pad, W1)
    h1p, dinv = _tc_scale(hp, degp.reshape(NC, NP4, 128))
    acc1 = _sc_aggregate(h1p.reshape(NPAD, H), src, dst, zeros_h)
    h2p = _tc_mid(acc1.reshape(NC, NP4, 128), dinv, w2p, b1p)
    acc2 = _sc_aggregate(h2p.reshape(NPAD, H), src, dst, zeros_h)
    outp = _tc_final(acc2.reshape(NC, NP4, 128), dinv, w3p, b2p, b3p)
    return outp.reshape(NPAD, 1)[:N]
